# Initial kernel scaffold; baseline (speedup 1.0000x reference)
import jax
import jax.numpy as jnp
from jax.experimental import pallas as pl


def kernel(x, edge_index, W1, att_src1, att_dst1, b1, W2, att_src2, att_dst2, b2, gamma, beta, Wout, bout):
    raise NotImplementedError("write your pallas kernel here")


# speedup vs baseline: 8.4014x; 8.4014x over previous
import functools

import jax
import jax.numpy as jnp
from jax import lax
from jax.experimental import pallas as pl
from jax.experimental.pallas import tpu as pltpu
from jax.experimental.pallas import tpu_sc as plsc

N = 10000
D = 128
E = 320000
NC = 2
NS = 16
EW = E // NS
K = 128
CH = 160
EWP = CH * K
NROW = 5120
AROWS_T = NROW // NS
NPAD = NC * NROW


def _leaky_exp(e):
    return jnp.exp(jnp.where(e >= 0, e, 0.2 * e))


def _tc1_body(x_ref, w_ref, asw_ref, adw_ref, h_ref, as_ref, ad_ref):
    h = jnp.dot(x_ref[...], w_ref[...], preferred_element_type=jnp.float32)
    h_ref[...] = h
    as_ref[...] = jnp.dot(h, asw_ref[...], preferred_element_type=jnp.float32)
    ad_ref[...] = jnp.dot(h, adw_ref[...], preferred_element_type=jnp.float32)


_tc1 = pl.pallas_call(
    _tc1_body,
    out_shape=[
        jax.ShapeDtypeStruct((N, D), jnp.float32),
        jax.ShapeDtypeStruct((N, 1), jnp.float32),
        jax.ShapeDtypeStruct((N, 1), jnp.float32),
    ],
)


def _combine(acc_pad, sst, a_s, a_d, h, b):
    ex = _leaky_exp(a_s + a_d)
    acc = acc_pad[:N, :] + ex * h
    ssum = jnp.sum(sst, axis=1, keepdims=True) + ex
    return acc / (ssum + 1e-16) + b


def _bn_elu(g, gamma, beta):
    mu = jnp.mean(g, axis=0, keepdims=True)
    var = jnp.mean((g - mu) ** 2, axis=0, keepdims=True)
    y = (g - mu) / jnp.sqrt(var + 1e-5) * gamma + beta
    return jnp.where(y > 0, y, jnp.exp(y) - 1.0)


def _tc2_body(acc_ref, sst_ref, as_ref, ad_ref, h_ref, b_ref, gamma_ref,
              beta_ref, w2_ref, asw2_ref, adw2_ref, h2_ref, as2_ref, ad2_ref):
    g = _combine(acc_ref[...], sst_ref[...], as_ref[...], ad_ref[...],
                 h_ref[...], b_ref[...])
    hin = _bn_elu(g, gamma_ref[...], beta_ref[...])
    h2 = jnp.dot(hin, w2_ref[...], preferred_element_type=jnp.float32)
    h2_ref[...] = h2
    as2_ref[...] = jnp.dot(h2, asw2_ref[...], preferred_element_type=jnp.float32)
    ad2_ref[...] = jnp.dot(h2, adw2_ref[...], preferred_element_type=jnp.float32)


_tc2 = pl.pallas_call(
    _tc2_body,
    out_shape=[
        jax.ShapeDtypeStruct((N, D), jnp.float32),
        jax.ShapeDtypeStruct((N, 1), jnp.float32),
        jax.ShapeDtypeStruct((N, 1), jnp.float32),
    ],
)


def _tc3_body(acc_ref, sst_ref, as_ref, ad_ref, h_ref, b_ref, gamma_ref,
              beta_ref, wout_ref, bout_ref, out_ref):
    g = _combine(acc_ref[...], sst_ref[...], as_ref[...], ad_ref[...],
                 h_ref[...], b_ref[...])
    hin = _bn_elu(g, gamma_ref[...], beta_ref[...])
    out_ref[...] = (jnp.dot(hin, wout_ref[...], preferred_element_type=jnp.float32)
                    + bout_ref[...])


_tc3 = pl.pallas_call(
    _tc3_body,
    out_shape=[jax.ShapeDtypeStruct((N, 64), jnp.float32)],
)


@functools.partial(
    pl.kernel,
    out_type=(
        jax.ShapeDtypeStruct((NPAD, D), jnp.float32),
        jax.ShapeDtypeStruct((NC, NS, N), jnp.float32),
    ),
    mesh=plsc.VectorSubcoreMesh(core_axis_name="c", subcore_axis_name="s",
                                num_cores=NC, num_subcores=NS),
    compiler_params=pltpu.CompilerParams(needs_layout_passes=False),
    scratch_types=[
        pltpu.VMEM((N,), jnp.float32),
        pltpu.VMEM((N,), jnp.float32),
        pltpu.VMEM((N,), jnp.float32),
        pltpu.VMEM((K, D), jnp.float32),
        pltpu.VMEM((2, K), jnp.int32),
        pltpu.VMEM((K,), jnp.float32),
        pltpu.VMEM_SHARED((NROW + 8, D), jnp.float32),
        pltpu.SemaphoreType.DMA,
    ],
)
def _sc_gat(h_hbm, as_hbm, ad_hbm, sd_hbm, zn_hbm, zrows_hbm,
            acc_hbm, sst_hbm,
            asv, adv, ssv, rows, sdbuf, exbuf, acc_sh, gsem):
    c = lax.axis_index("c")
    s = lax.axis_index("s")

    pltpu.sync_copy(as_hbm, asv)
    pltpu.sync_copy(ad_hbm, adv)
    pltpu.sync_copy(zn_hbm, ssv)
    pltpu.sync_copy(zrows_hbm, acc_sh.at[pl.ds(s * AROWS_T, AROWS_T), :])

    @pl.when(s == 0)
    def _():
        pltpu.sync_copy(zrows_hbm.at[pl.ds(0, 8), :],
                        acc_sh.at[pl.ds(NROW, 8), :])

    plsc.subcore_barrier()

    cbase = jnp.zeros((16,), jnp.int32) + c * NROW

    def chunk(j, carry):
        pltpu.sync_copy(sd_hbm.at[s, j], sdbuf)

        cp = pltpu.async_copy(h_hbm.at[sdbuf.at[0]], rows, gsem)

        for l in range(K // 16):
            sv = sdbuf[0, pl.ds(l * 16, 16)]
            dv = sdbuf[1, pl.ds(l * 16, 16)]
            e = plsc.load_gather(asv, [sv]) + plsc.load_gather(adv, [dv])
            ex = _leaky_exp(e)
            valid = (j * K + l * 16 + lax.iota(jnp.int32, 16)) < EW
            ex = jnp.where(valid, ex, 0.0)
            exbuf[pl.ds(l * 16, 16)] = ex
            plsc.addupdate_scatter(ssv, [dv], ex)
            dl = dv - cbase
            mine = (dl >= 0) & (dl < NROW)
            sdbuf[1, pl.ds(l * 16, 16)] = jnp.where(mine, dl, NROW)

        cp.wait()

        def scale_one(l, cc):
            exb = plsc.load_gather(exbuf, [jnp.zeros((16,), jnp.int32) + l])
            for g in range(D // 16):
                rows[l, pl.ds(g * 16, 16)] = rows[l, pl.ds(g * 16, 16)] * exb
            return cc

        lax.fori_loop(0, K, scale_one, 0)
        pltpu.sync_copy(rows, acc_sh.at[sdbuf.at[1]], add=True)
        return carry

    lax.fori_loop(0, CH, chunk, 0)
    plsc.subcore_barrier()

    pltpu.sync_copy(ssv, sst_hbm.at[c, s])
    pltpu.sync_copy(acc_sh.at[pl.ds(s * AROWS_T, AROWS_T), :],
                    acc_hbm.at[pl.ds(c * NROW + s * AROWS_T, AROWS_T), :])


def kernel(x, edge_index, W1, att_src1, att_dst1, b1, W2, att_src2, att_dst2,
           b2, gamma, beta, Wout, bout):
    ei = edge_index.astype(jnp.int32)
    src2 = jnp.pad(ei[0].reshape(NS, EW), ((0, 0), (0, EWP - EW))).reshape(NS, CH, K)
    dst2 = jnp.pad(ei[1].reshape(NS, EW), ((0, 0), (0, EWP - EW))).reshape(NS, CH, K)
    sd = jnp.stack([src2, dst2], axis=2)
    zn = jnp.zeros((N,), jnp.float32)
    zrows = jnp.zeros((AROWS_T, D), jnp.float32)

    b1r = b1.reshape(1, D)
    b2r = b2.reshape(1, D)
    gr = gamma.reshape(1, D)
    br = beta.reshape(1, D)

    h1, as1, ad1 = _tc1(x, W1, att_src1.reshape(1, D).T, att_dst1.reshape(1, D).T)
    acc1, ssp1 = _sc_gat(h1, as1.reshape(N), ad1.reshape(N), sd, zn, zrows)
    sst1 = jnp.swapaxes(ssp1[0], 0, 1)
    h2, as2, ad2 = _tc2(acc1, sst1, as1, ad1, h1, b1r, gr, br, W2,
                        att_src2.reshape(1, D).T, att_dst2.reshape(1, D).T)
    acc2, ssp2 = _sc_gat(h2, as2.reshape(N), ad2.reshape(N), sd, zn, zrows)
    sst2 = jnp.swapaxes(ssp2[0], 0, 1)
    (out,) = _tc3(acc2, sst2, as2, ad2, h2, b2r, gr, br, Wout,
                  bout.reshape(1, 64))
    return out


# speedup vs baseline: 15.7182x; 1.8709x over previous
import functools

import jax
import jax.numpy as jnp
from jax import lax
from jax.experimental import pallas as pl
from jax.experimental.pallas import tpu as pltpu
from jax.experimental.pallas import tpu_sc as plsc

N = 10000
D = 128
E = 320000
NC = 2
NS = 16
NW = NC * NS
EW = E // NW
K = 128
CH = 80
EWP = CH * K
NPAD = 10240
ROWS_T = NPAD // NS


def _leaky_exp(e):
    return jnp.exp(jnp.where(e >= 0, e, 0.2 * e))


def _tc1_body(x_ref, w_ref, asw_ref, adw_ref, h_ref, as_ref, ad_ref):
    h = jnp.dot(x_ref[...], w_ref[...], preferred_element_type=jnp.float32)
    h_ref[...] = h
    as_ref[...] = jnp.dot(h, asw_ref[...], preferred_element_type=jnp.float32)
    ad_ref[...] = jnp.dot(h, adw_ref[...], preferred_element_type=jnp.float32)


_tc1 = pl.pallas_call(
    _tc1_body,
    out_shape=[
        jax.ShapeDtypeStruct((N, D), jnp.float32),
        jax.ShapeDtypeStruct((N, 1), jnp.float32),
        jax.ShapeDtypeStruct((N, 1), jnp.float32),
    ],
)


def _combine(accp, sst, a_s, a_d, h, b):
    ex = _leaky_exp(a_s + a_d)
    acc = accp[0, :N, :] + accp[1, :N, :] + ex * h
    ssum = jnp.sum(sst, axis=1, keepdims=True) + ex
    return acc / (ssum + 1e-16) + b


def _bn_elu(g, gamma, beta):
    mu = jnp.mean(g, axis=0, keepdims=True)
    var = jnp.mean((g - mu) ** 2, axis=0, keepdims=True)
    y = (g - mu) / jnp.sqrt(var + 1e-5) * gamma + beta
    return jnp.where(y > 0, y, jnp.exp(y) - 1.0)


def _tc2_body(acc_ref, sst_ref, as_ref, ad_ref, h_ref, b_ref, gamma_ref,
              beta_ref, w2_ref, asw2_ref, adw2_ref, h2_ref, as2_ref, ad2_ref):
    g = _combine(acc_ref[...], sst_ref[...], as_ref[...], ad_ref[...],
                 h_ref[...], b_ref[...])
    hin = _bn_elu(g, gamma_ref[...], beta_ref[...])
    h2 = jnp.dot(hin, w2_ref[...], preferred_element_type=jnp.float32)
    h2_ref[...] = h2
    as2_ref[...] = jnp.dot(h2, asw2_ref[...], preferred_element_type=jnp.float32)
    ad2_ref[...] = jnp.dot(h2, adw2_ref[...], preferred_element_type=jnp.float32)


_tc2 = pl.pallas_call(
    _tc2_body,
    out_shape=[
        jax.ShapeDtypeStruct((N, D), jnp.float32),
        jax.ShapeDtypeStruct((N, 1), jnp.float32),
        jax.ShapeDtypeStruct((N, 1), jnp.float32),
    ],
)


def _tc3_body(acc_ref, sst_ref, as_ref, ad_ref, h_ref, b_ref, gamma_ref,
              beta_ref, wout_ref, bout_ref, out_ref):
    g = _combine(acc_ref[...], sst_ref[...], as_ref[...], ad_ref[...],
                 h_ref[...], b_ref[...])
    hin = _bn_elu(g, gamma_ref[...], beta_ref[...])
    out_ref[...] = (jnp.dot(hin, wout_ref[...], preferred_element_type=jnp.float32)
                    + bout_ref[...])


_tc3 = pl.pallas_call(
    _tc3_body,
    out_shape=[jax.ShapeDtypeStruct((N, 64), jnp.float32)],
)


@functools.partial(
    pl.kernel,
    out_type=(
        jax.ShapeDtypeStruct((NC, NPAD, D), jnp.float32),
        jax.ShapeDtypeStruct((NC, NS, N), jnp.float32),
    ),
    mesh=plsc.VectorSubcoreMesh(core_axis_name="c", subcore_axis_name="s",
                                num_cores=NC, num_subcores=NS),
    compiler_params=pltpu.CompilerParams(needs_layout_passes=False),
    scratch_types=[
        pltpu.VMEM((N,), jnp.float32),
        pltpu.VMEM((N,), jnp.float32),
        pltpu.VMEM((N,), jnp.float32),
        pltpu.VMEM((K, D), jnp.float32),
        pltpu.VMEM((2, K), jnp.int32),
        pltpu.VMEM((K,), jnp.float32),
        pltpu.VMEM_SHARED((NPAD, D), jnp.float32),
        pltpu.SemaphoreType.DMA,
    ],
)
def _sc_gat(h_hbm, as_hbm, ad_hbm, sd_hbm, zn_hbm, zrows_hbm,
            acc_hbm, sst_hbm,
            asv, adv, ssv, rows, sdbuf, exbuf, acc_sh, gsem):
    c = lax.axis_index("c")
    s = lax.axis_index("s")
    w = c * NS + s

    pltpu.sync_copy(as_hbm, asv)
    pltpu.sync_copy(ad_hbm, adv)
    pltpu.sync_copy(zn_hbm, ssv)
    pltpu.sync_copy(zrows_hbm, acc_sh.at[pl.ds(s * ROWS_T, ROWS_T), :])
    plsc.subcore_barrier()

    def chunk(j, carry):
        pltpu.sync_copy(sd_hbm.at[w, j], sdbuf)

        cp = pltpu.async_copy(h_hbm.at[sdbuf.at[0]], rows, gsem)

        for l in range(K // 16):
            sv = sdbuf[0, pl.ds(l * 16, 16)]
            dv = sdbuf[1, pl.ds(l * 16, 16)]
            e = plsc.load_gather(asv, [sv]) + plsc.load_gather(adv, [dv])
            ex = _leaky_exp(e)
            valid = (j * K + l * 16 + lax.iota(jnp.int32, 16)) < EW
            ex = jnp.where(valid, ex, 0.0)
            exbuf[pl.ds(l * 16, 16)] = ex
            plsc.addupdate_scatter(ssv, [dv], ex)

        cp.wait()

        def scale_one(l, cc):
            exb = plsc.load_gather(exbuf, [jnp.zeros((16,), jnp.int32) + l])
            for g in range(D // 16):
                rows[l, pl.ds(g * 16, 16)] = rows[l, pl.ds(g * 16, 16)] * exb
            return cc

        lax.fori_loop(0, K, scale_one, 0)
        pltpu.sync_copy(rows, acc_sh.at[sdbuf.at[1]], add=True)
        return carry

    lax.fori_loop(0, CH, chunk, 0)
    plsc.subcore_barrier()

    pltpu.sync_copy(ssv, sst_hbm.at[c, s])
    pltpu.sync_copy(acc_sh.at[pl.ds(s * ROWS_T, ROWS_T), :],
                    acc_hbm.at[c, pl.ds(s * ROWS_T, ROWS_T), :])


def kernel(x, edge_index, W1, att_src1, att_dst1, b1, W2, att_src2, att_dst2,
           b2, gamma, beta, Wout, bout):
    ei = edge_index.astype(jnp.int32)
    src2 = jnp.pad(ei[0].reshape(NW, EW), ((0, 0), (0, EWP - EW))).reshape(NW, CH, K)
    dst2 = jnp.pad(ei[1].reshape(NW, EW), ((0, 0), (0, EWP - EW))).reshape(NW, CH, K)
    sd = jnp.stack([src2, dst2], axis=2)
    zn = jnp.zeros((N,), jnp.float32)
    zrows = jnp.zeros((ROWS_T, D), jnp.float32)

    b1r = b1.reshape(1, D)
    b2r = b2.reshape(1, D)
    gr = gamma.reshape(1, D)
    br = beta.reshape(1, D)

    h1, as1, ad1 = _tc1(x, W1, att_src1.reshape(1, D).T, att_dst1.reshape(1, D).T)
    acc1, ssp1 = _sc_gat(h1, as1.reshape(N), ad1.reshape(N), sd, zn, zrows)
    sst1 = jnp.swapaxes(ssp1.reshape(NW, N), 0, 1)
    h2, as2, ad2 = _tc2(acc1, sst1, as1, ad1, h1, b1r, gr, br, W2,
                        att_src2.reshape(1, D).T, att_dst2.reshape(1, D).T)
    acc2, ssp2 = _sc_gat(h2, as2.reshape(N), ad2.reshape(N), sd, zn, zrows)
    sst2 = jnp.swapaxes(ssp2.reshape(NW, N), 0, 1)
    (out,) = _tc3(acc2, sst2, as2, ad2, h2, b2r, gr, br, Wout,
                  bout.reshape(1, 64))
    return out


# speedup vs baseline: 19.1916x; 1.2210x over previous
import functools

import jax
import jax.numpy as jnp
from jax import lax
from jax.experimental import pallas as pl
from jax.experimental.pallas import tpu as pltpu
from jax.experimental.pallas import tpu_sc as plsc

N = 10000
D = 128
E = 320000
NC = 2
NS = 16
NW = NC * NS
EW = E // NW
K = 64
CH = 160
EWP = CH * K
NPAD = 10240
ROWS_T = NPAD // NS


def _leaky_exp(e):
    return jnp.exp(jnp.where(e >= 0, e, 0.2 * e))


def _tc1_body(x_ref, w_ref, asw_ref, adw_ref, h_ref, as_ref, ad_ref):
    h = jnp.dot(x_ref[...], w_ref[...], preferred_element_type=jnp.float32)
    h_ref[...] = h
    as_ref[...] = jnp.dot(h, asw_ref[...], preferred_element_type=jnp.float32)
    ad_ref[...] = jnp.dot(h, adw_ref[...], preferred_element_type=jnp.float32)


_tc1 = pl.pallas_call(
    _tc1_body,
    out_shape=[
        jax.ShapeDtypeStruct((N, D), jnp.float32),
        jax.ShapeDtypeStruct((N, 1), jnp.float32),
        jax.ShapeDtypeStruct((N, 1), jnp.float32),
    ],
)


def _combine(accp, sst, a_s, a_d, h, b):
    ex = _leaky_exp(a_s + a_d)
    acc = accp[0, :N, :] + accp[1, :N, :] + ex * h
    ssum = jnp.sum(sst, axis=1, keepdims=True) + ex
    return acc / (ssum + 1e-16) + b


def _bn_elu(g, gamma, beta):
    mu = jnp.mean(g, axis=0, keepdims=True)
    var = jnp.mean((g - mu) ** 2, axis=0, keepdims=True)
    y = (g - mu) / jnp.sqrt(var + 1e-5) * gamma + beta
    return jnp.where(y > 0, y, jnp.exp(y) - 1.0)


def _tc2_body(acc_ref, sst_ref, as_ref, ad_ref, h_ref, b_ref, gamma_ref,
              beta_ref, w2_ref, asw2_ref, adw2_ref, h2_ref, as2_ref, ad2_ref):
    g = _combine(acc_ref[...], sst_ref[...], as_ref[...], ad_ref[...],
                 h_ref[...], b_ref[...])
    hin = _bn_elu(g, gamma_ref[...], beta_ref[...])
    h2 = jnp.dot(hin, w2_ref[...], preferred_element_type=jnp.float32)
    h2_ref[...] = h2
    as2_ref[...] = jnp.dot(h2, asw2_ref[...], preferred_element_type=jnp.float32)
    ad2_ref[...] = jnp.dot(h2, adw2_ref[...], preferred_element_type=jnp.float32)


_tc2 = pl.pallas_call(
    _tc2_body,
    out_shape=[
        jax.ShapeDtypeStruct((N, D), jnp.float32),
        jax.ShapeDtypeStruct((N, 1), jnp.float32),
        jax.ShapeDtypeStruct((N, 1), jnp.float32),
    ],
)


def _tc3_body(acc_ref, sst_ref, as_ref, ad_ref, h_ref, b_ref, gamma_ref,
              beta_ref, wout_ref, bout_ref, out_ref):
    g = _combine(acc_ref[...], sst_ref[...], as_ref[...], ad_ref[...],
                 h_ref[...], b_ref[...])
    hin = _bn_elu(g, gamma_ref[...], beta_ref[...])
    out_ref[...] = (jnp.dot(hin, wout_ref[...], preferred_element_type=jnp.float32)
                    + bout_ref[...])


_tc3 = pl.pallas_call(
    _tc3_body,
    out_shape=[jax.ShapeDtypeStruct((N, 64), jnp.float32)],
)


@functools.partial(
    pl.kernel,
    out_type=(
        jax.ShapeDtypeStruct((NC, NPAD, D), jnp.float32),
        jax.ShapeDtypeStruct((NC, NS, N), jnp.float32),
    ),
    mesh=plsc.VectorSubcoreMesh(core_axis_name="c", subcore_axis_name="s",
                                num_cores=NC, num_subcores=NS),
    compiler_params=pltpu.CompilerParams(needs_layout_passes=False),
    scratch_types=[
        pltpu.VMEM((N,), jnp.float32),
        pltpu.VMEM((N,), jnp.float32),
        pltpu.VMEM((N,), jnp.float32),
        pltpu.VMEM((K, D), jnp.float32),
        pltpu.VMEM((K, D), jnp.float32),
        pltpu.VMEM((2, K), jnp.int32),
        pltpu.VMEM((2, K), jnp.int32),
        pltpu.VMEM((2, K), jnp.int32),
        pltpu.VMEM((2, K), jnp.int32),
        pltpu.VMEM((K,), jnp.float32),
        pltpu.VMEM((K,), jnp.float32),
        pltpu.VMEM_SHARED((NPAD, D), jnp.float32),
        pltpu.SemaphoreType.DMA,
        pltpu.SemaphoreType.DMA,
        pltpu.SemaphoreType.DMA,
        pltpu.SemaphoreType.DMA,
        pltpu.SemaphoreType.DMA,
        pltpu.SemaphoreType.DMA,
        pltpu.SemaphoreType.DMA,
        pltpu.SemaphoreType.DMA,
    ],
)
def _sc_gat(h_hbm, as_hbm, ad_hbm, sd_hbm, zn_hbm, zrows_hbm,
            acc_hbm, sst_hbm,
            asv, adv, ssv, rows0, rows1, sdbuf0, sdbuf1, sdbuf2, sdbuf3,
            exbuf0, exbuf1, acc_sh, gsem0, gsem1, ssem0, ssem1,
            isem0, isem1, isem2, isem3):
    c = lax.axis_index("c")
    s = lax.axis_index("s")
    w = c * NS + s

    pltpu.sync_copy(as_hbm, asv)
    pltpu.sync_copy(ad_hbm, adv)
    pltpu.sync_copy(zn_hbm, ssv)
    pltpu.sync_copy(zrows_hbm, acc_sh.at[pl.ds(s * ROWS_T, ROWS_T), :])
    plsc.subcore_barrier()

    sdbufs = (sdbuf0, sdbuf1, sdbuf2, sdbuf3)
    rowss = (rows0, rows1)
    exbufs = (exbuf0, exbuf1)
    gsems = (gsem0, gsem1)
    ssems = (ssem0, ssem1)
    isems = (isem0, isem1, isem2, isem3)
    CHQ = CH // 4

    pltpu.async_copy(sd_hbm.at[w, 0], sdbuf0, isem0)
    pltpu.async_copy(sd_hbm.at[w, 1], sdbuf1, isem1)
    pltpu.make_async_copy(sd_hbm.at[w, 0], sdbuf0, isem0).wait()
    pltpu.async_copy(h_hbm.at[sdbuf0.at[0]], rows0, gsem0)

    def one_chunk(ip, q):
        j = ip * 4 + q
        b = q % 2
        rows, exbuf, sdbuf = rowss[b], exbufs[b], sdbufs[q]

        pltpu.make_async_copy(h_hbm.at[sdbuf.at[0]], rows, gsems[b]).wait()

        for l in range(K // 16):
            sv = sdbuf[0, pl.ds(l * 16, 16)]
            dv = sdbuf[1, pl.ds(l * 16, 16)]
            e = plsc.load_gather(asv, [sv]) + plsc.load_gather(adv, [dv])
            ex = _leaky_exp(e)
            valid = (j * K + l * 16 + lax.iota(jnp.int32, 16)) < EW
            ex = jnp.where(valid, ex, 0.0)
            exbuf[pl.ds(l * 16, 16)] = ex
            plsc.addupdate_scatter(ssv, [dv], ex)

        if q == 0:
            @pl.when(ip > 0)
            def _():
                pltpu.make_async_copy(rowss[1 - b], acc_sh.at[sdbufs[3].at[1]],
                                      ssems[1 - b]).wait()
        else:
            pltpu.make_async_copy(rowss[1 - b], acc_sh.at[sdbufs[q - 1].at[1]],
                                  ssems[1 - b]).wait()

        qn = (q + 2) % 4
        if q < 2:
            pltpu.async_copy(sd_hbm.at[w, j + 2], sdbufs[qn], isems[qn])
        else:
            @pl.when(ip < CHQ - 1)
            def _():
                pltpu.async_copy(sd_hbm.at[w, j + 2], sdbufs[qn], isems[qn])

        q1 = (q + 1) % 4
        if q < 3:
            pltpu.make_async_copy(sd_hbm.at[w, j + 1], sdbufs[q1], isems[q1]).wait()
            pltpu.async_copy(h_hbm.at[sdbufs[q1].at[0]], rowss[1 - b], gsems[1 - b])
        else:
            @pl.when(ip < CHQ - 1)
            def _():
                pltpu.make_async_copy(sd_hbm.at[w, j + 1], sdbufs[q1],
                                      isems[q1]).wait()
                pltpu.async_copy(h_hbm.at[sdbufs[q1].at[0]], rowss[1 - b],
                                 gsems[1 - b])

        def scale_two(i, cc):
            l = i * 2
            exa = plsc.load_gather(exbuf, [jnp.zeros((16,), jnp.int32) + l])
            exb = plsc.load_gather(exbuf, [jnp.zeros((16,), jnp.int32) + (l + 1)])
            for g in range(D // 16):
                rows[l, pl.ds(g * 16, 16)] = rows[l, pl.ds(g * 16, 16)] * exa
            for g in range(D // 16):
                rows[l + 1, pl.ds(g * 16, 16)] = rows[l + 1, pl.ds(g * 16, 16)] * exb
            return cc

        lax.fori_loop(0, K // 2, scale_two, 0)
        pltpu.async_copy(rows, acc_sh.at[sdbuf.at[1]], ssems[b], add=True)

    def quad(ip, carry):
        for q in range(4):
            one_chunk(ip, q)
        return carry

    lax.fori_loop(0, CHQ, quad, 0)
    pltpu.make_async_copy(rows1, acc_sh.at[sdbuf3.at[1]], ssem1).wait()
    plsc.subcore_barrier()

    pltpu.sync_copy(ssv, sst_hbm.at[c, s])
    pltpu.sync_copy(acc_sh.at[pl.ds(s * ROWS_T, ROWS_T), :],
                    acc_hbm.at[c, pl.ds(s * ROWS_T, ROWS_T), :])


def kernel(x, edge_index, W1, att_src1, att_dst1, b1, W2, att_src2, att_dst2,
           b2, gamma, beta, Wout, bout):
    ei = edge_index.astype(jnp.int32)
    src2 = jnp.pad(ei[0].reshape(NW, EW), ((0, 0), (0, EWP - EW))).reshape(NW, CH, K)
    dst2 = jnp.pad(ei[1].reshape(NW, EW), ((0, 0), (0, EWP - EW))).reshape(NW, CH, K)
    sd = jnp.stack([src2, dst2], axis=2)
    zn = jnp.zeros((N,), jnp.float32)
    zrows = jnp.zeros((ROWS_T, D), jnp.float32)

    b1r = b1.reshape(1, D)
    b2r = b2.reshape(1, D)
    gr = gamma.reshape(1, D)
    br = beta.reshape(1, D)

    h1, as1, ad1 = _tc1(x, W1, att_src1.reshape(1, D).T, att_dst1.reshape(1, D).T)
    acc1, ssp1 = _sc_gat(h1, as1.reshape(N), ad1.reshape(N), sd, zn, zrows)
    sst1 = jnp.swapaxes(ssp1.reshape(NW, N), 0, 1)
    h2, as2, ad2 = _tc2(acc1, sst1, as1, ad1, h1, b1r, gr, br, W2,
                        att_src2.reshape(1, D).T, att_dst2.reshape(1, D).T)
    acc2, ssp2 = _sc_gat(h2, as2.reshape(N), ad2.reshape(N), sd, zn, zrows)
    sst2 = jnp.swapaxes(ssp2.reshape(NW, N), 0, 1)
    (out,) = _tc3(acc2, sst2, as2, ad2, h2, b2r, gr, br, Wout,
                  bout.reshape(1, 64))
    return out
